# Initial kernel scaffold; baseline (speedup 1.0000x reference)
import jax
import jax.numpy as jnp
from jax.experimental import pallas as pl


def kernel(points, inlier_thre, score_thresh):
    raise NotImplementedError("write your pallas kernel here")


# speedup vs baseline: 14.5996x; 14.5996x over previous
import jax
import jax.numpy as jnp
from jax import lax
from jax.experimental import pallas as pl

_M = 256
_Q = 100


def _body(pts_ref, it_ref, st_ref, score_ref, cf_ref, tf_ref, th_ref):
    f32 = jnp.float32
    pts = pts_ref[...]
    a1 = pts[:, 0:3]
    a2 = pts[:, 3:6]

    row = lax.broadcasted_iota(jnp.int32, (_M, _M), 0)
    col = lax.broadcasted_iota(jnp.int32, (_M, _M), 1)
    eye = (row == col).astype(f32)

    def gram(a):
        ab = a.astype(jnp.bfloat16)
        return lax.dot_general(ab, ab, (((1,), (1,)), ((), ())),
                               preferred_element_type=f32)

    def col2row(v):
        return jnp.sum(eye * v, axis=0, keepdims=True)

    def dist(a, G):
        n_col = jnp.sum(a * a, axis=1, keepdims=True)
        n_row = col2row(n_col)
        sq = n_col + n_row - 2.0 * G
        return jnp.sqrt(jnp.maximum(sq, 0.0) + 1e-12)

    d1 = dist(a1, gram(a1))
    d2 = dist(a2, gram(a2))
    dmat = jnp.abs(d1 - d2)

    it = it_ref[0, 0]
    st = st_ref[0, 0]
    sc = jnp.exp(-(dmat * dmat) / (2.0 * it * it))
    sc = jnp.where(sc < st, 0.0, sc)
    scb = sc.astype(jnp.bfloat16)
    sq2 = lax.dot_general(scb, scb, (((1,), (0,)), ((), ())),
                          preferred_element_type=f32)
    s = sc * sq2
    score_ref[...] = s

    snd = s * (1.0 - eye)
    degree = jnp.sum((snd != 0.0).astype(f32), axis=1, keepdims=True)

    c = jnp.where(snd > 0.0,
                  jnp.exp(jnp.log(jnp.where(snd > 0.0, snd, 1.0)) / 3.0),
                  0.0)
    cc = lax.dot_general(c, c, (((1,), (0,)), ((), ())),
                         preferred_element_type=f32,
                         precision=lax.Precision.HIGHEST)
    wijk = 0.5 * jnp.sum(cc * c, axis=1, keepdims=True)

    invalid = degree <= 1.0
    deg = jnp.where(invalid, 0.0, degree)
    f1 = jnp.where(invalid, 0.0, wijk)
    f2 = deg * (deg - 1.0) * 0.5
    sum_fenzi = jnp.sum(f1)
    sum_fenmu = jnp.sum(f2) + 1e-10
    f2 = jnp.where(invalid, 1.0, f2)
    cf = f1 / f2
    cf_ref[...] = cf
    tf_ref[...] = (sum_fenzi / sum_fenmu).reshape(1, 1)

    maxv = jnp.max(cf)
    minv = jnp.min(cf)
    step = (maxv - minv) / _Q
    ids = (cf / jnp.where(step == 0.0, 1.0, step)).astype(jnp.int32)
    ids = jnp.where(ids >= _Q, _Q - 1, ids)
    valid = ids >= 0
    qs = lax.broadcasted_iota(jnp.int32, (_M, _Q), 1)
    le = (ids <= qs) & valid
    n1 = jnp.sum(le.astype(f32), axis=0, keepdims=True)
    fore = jnp.sum(jnp.where(le, cf, 0.0), axis=0, keepdims=True)
    total = jnp.sum(cf)
    n2 = f32(_M) - n1
    m1 = fore / jnp.where(n1 == 0.0, 1.0, n1)
    m2 = (total - fore) / jnp.where(n2 == 0.0, 1.0, n2)
    sb = n1 * n2 * (m1 - m2) ** 2
    sb = jnp.where((n1 > 0.0) & (n2 > 0.0), sb, -jnp.inf)
    sbmax = jnp.max(sb)
    qidx = lax.broadcasted_iota(jnp.int32, (1, _Q), 1)
    best = jnp.min(jnp.where(sb == sbmax, qidx, _Q)).astype(f32)
    tval = best * (maxv - minv) / _Q
    th_ref[...] = jnp.where(sbmax > -1000.0, tval, 0.0).reshape(1, 1)


def kernel(points, inlier_thre, score_thresh):
    pts = points.reshape(_M, 6)
    it = jnp.asarray(inlier_thre, jnp.float32).reshape(1, 1)
    st = jnp.asarray(score_thresh, jnp.float32).reshape(1, 1)
    out_shapes = (
        jax.ShapeDtypeStruct((_M, _M), jnp.float32),
        jax.ShapeDtypeStruct((_M, 1), jnp.float32),
        jax.ShapeDtypeStruct((1, 1), jnp.float32),
        jax.ShapeDtypeStruct((1, 1), jnp.float32),
    )
    s, cf, tf, th = pl.pallas_call(
        _body,
        out_shape=out_shapes,
    )(pts, it, st)
    return (s.reshape(1, _M, _M), cf.reshape(_M),
            tf.reshape(()), th.reshape(()))
